# baseline (speedup 1.0000x reference)
import jax
import jax.numpy as jnp
from jax import lax
from jax.experimental import pallas as pl
from jax.experimental.pallas import tpu as pltpu
from jax.experimental.pallas import tpu_sc as plsc

VOCAB = 1000000
EMB = 64
NTOK = 16384
NPOS = 50
W2V = VOCAB // 2

NC, NS = 2, 16
NW = NC * NS
T_PER_W = NTOK // NW
NTB = T_PER_W // 128
NBLK = NPOS * NTB


def _emb_kernel(ids_hbm, w2_hbm, out_hbm, idx_v, pbuf, rbuf, bbuf, gsem,
                osem):
    c = lax.axis_index("c")
    s = lax.axis_index("s")
    wid = c * NS + s
    t_base = wid * T_PER_W

    pltpu.sync_copy(ids_hbm.at[wid], idx_v)

    lane = lax.broadcasted_iota(jnp.int32, (16,), 0)

    def block(j):
        k = j // NTB
        t0 = t_base + (j % NTB) * 128

        for g in range(8):
            pbuf[pl.ds(g * 16, 16)] = jax.lax.shift_right_logical(
                idx_v[j, pl.ds(g * 16, 16)], 1)

        pltpu.async_copy(w2_hbm.at[pbuf], rbuf, gsem).wait()

        rows = []
        cols = []
        for g in range(8):
            raw = idx_v[j, pl.ds(g * 16, 16)]
            rows.append(g * 16 + lane)
            cols.append((raw & 1) * EMB)

        def erow(e):
            for g in range(8):
                bbuf[e, pl.ds(g * 16, 16)] = plsc.load_gather(
                    rbuf, [rows[g], cols[g] + e])

        pl.loop(0, EMB)(erow)

        pltpu.sync_copy(bbuf, out_hbm.at[k, :, pl.ds(t0, 128)])

    pl.loop(0, NBLK)(block)


def kernel(input_ids, weight):
    w2 = weight.reshape(W2V, 128)
    ids_blk = (input_ids.T.reshape(NPOS, NW, NTB, 128)
               .transpose(1, 0, 2, 3).reshape(NW, NBLK, 128))
    mesh = plsc.VectorSubcoreMesh(core_axis_name="c", subcore_axis_name="s")
    out = pl.kernel(
        _emb_kernel,
        mesh=mesh,
        compiler_params=pltpu.CompilerParams(use_tc_tiling_on_sc=True,
                                             needs_layout_passes=False),
        out_type=jax.ShapeDtypeStruct((NPOS, EMB, NTOK), jnp.float32),
        scratch_types=[
            pltpu.VMEM((NBLK, 128), jnp.int32),
            pltpu.VMEM((128,), jnp.int32),
            pltpu.VMEM((128, 128), jnp.float32),
            pltpu.VMEM((EMB, 128), jnp.float32),
            pltpu.SemaphoreType.DMA,
            pltpu.SemaphoreType.DMA,
        ],
    )(ids_blk, w2)
    return jnp.transpose(out, (2, 0, 1))


# speedup vs baseline: 1.1996x; 1.1996x over previous
import jax
import jax.numpy as jnp
from jax import lax
from jax.experimental import pallas as pl
from jax.experimental.pallas import tpu as pltpu
from jax.experimental.pallas import tpu_sc as plsc

VOCAB = 1000000
EMB = 64
NTOK = 16384
NPOS = 50
W2V = VOCAB // 2

NC, NS = 2, 16
NW = NC * NS
T_PER_W = NTOK // NW
NTB = T_PER_W // 128
NBLK = NPOS * NTB


def _emb_kernel(ids_hbm, w2_hbm, out_hbm, idx_v, pbuf, rbuf, bbuf, gsems,
                osems):
    c = lax.axis_index("c")
    s = lax.axis_index("s")
    wid = c * NS + s
    t_base = wid * T_PER_W

    pltpu.sync_copy(ids_hbm.at[wid], idx_v)

    lane = lax.broadcasted_iota(jnp.int32, (16,), 0)

    def prep_and_fire(j, par):
        for g in range(8):
            pbuf[par, pl.ds(g * 16, 16)] = jax.lax.shift_right_logical(
                idx_v[j, pl.ds(g * 16, 16)], 1)
        pltpu.async_copy(w2_hbm.at[pbuf.at[par]], rbuf.at[par],
                         gsems.at[par])

    def out_slot(j):
        return out_hbm.at[j // NTB, :, pl.ds(t_base + (j % NTB) * 128, 128)]

    def process(j, par):
        pltpu.make_async_copy(w2_hbm.at[pbuf.at[par]], rbuf.at[par],
                              gsems.at[par]).wait()
        rows = []
        cols = []
        for g in range(8):
            raw = idx_v[j, pl.ds(g * 16, 16)]
            rows.append(g * 16 + lane)
            cols.append((raw & 1) * EMB)

        rb = rbuf.at[par]
        bb = bbuf.at[par]

        def erow(eg):
            for e8 in range(8):
                e = eg * 8 + e8
                for g in range(8):
                    bb[e, pl.ds(g * 16, 16)] = plsc.load_gather(
                        rb, [rows[g], cols[g] + e])

        pl.loop(0, 8)(erow)
        pltpu.async_copy(bb, out_slot(j), osems.at[par])

    def wait_store(j, par):
        pltpu.make_async_copy(bbuf.at[par], out_slot(j), osems.at[par]).wait()

    prep_and_fire(0, 0)

    def outer(m):
        for par in range(2):
            j = m * 2 + par

            @pl.when(j + 1 < NBLK)
            def _():
                prep_and_fire(j + 1, 1 - par)

            @pl.when(j >= 2)
            def _():
                wait_store(j - 2, par)

            process(j, par)

    pl.loop(0, NBLK // 2)(outer)
    wait_store(NBLK - 2, 0)
    wait_store(NBLK - 1, 1)


def kernel(input_ids, weight):
    w2 = weight.reshape(W2V, 128)
    ids_blk = (input_ids.T.reshape(NPOS, NW, NTB, 128)
               .transpose(1, 0, 2, 3).reshape(NW, NBLK, 128))
    mesh = plsc.VectorSubcoreMesh(core_axis_name="c", subcore_axis_name="s")
    out = pl.kernel(
        _emb_kernel,
        mesh=mesh,
        compiler_params=pltpu.CompilerParams(use_tc_tiling_on_sc=True,
                                             needs_layout_passes=False),
        out_type=jax.ShapeDtypeStruct((NPOS, EMB, NTOK), jnp.float32),
        scratch_types=[
            pltpu.VMEM((NBLK, 128), jnp.int32),
            pltpu.VMEM((2, 128), jnp.int32),
            pltpu.VMEM((2, 128, 128), jnp.float32),
            pltpu.VMEM((2, EMB, 128), jnp.float32),
            pltpu.SemaphoreType.DMA((2,)),
            pltpu.SemaphoreType.DMA((2,)),
        ],
    )(ids_blk, w2)
    return jnp.transpose(out, (2, 0, 1))
